# Initial kernel scaffold; baseline (speedup 1.0000x reference)
import jax
import jax.numpy as jnp
from jax.experimental import pallas as pl


def kernel(visual_codes, audio_codes, visual_traces, audio_traces, association):
    raise NotImplementedError("write your pallas kernel here")


# speedup vs baseline: 3.1741x; 3.1741x over previous
import functools

import jax
import jax.numpy as jnp
from jax import lax
from jax.experimental import pallas as pl
from jax.experimental.pallas import tpu as pltpu
from jax.experimental.pallas import tpu_sc as plsc

_NUM_BINS = 128
_N_SCALES = 4
_LAM_V = (0.97, 0.95, 0.9, 0.85)
_LAM_A = (0.97, 0.95, 0.9, 0.85)
_W_SCALE = 1.0 / _N_SCALES
_WD = 1e-05
_CLIP = 0.01

_NC = 2
_NS = 16
_NW = _NC * _NS
_L = 16


@functools.lru_cache(maxsize=None)
def _make_sc_hist(n):
    per = n // _NW
    mesh = plsc.VectorSubcoreMesh(core_axis_name="c", subcore_axis_name="s")

    @functools.partial(
        pl.kernel,
        out_type=jax.ShapeDtypeStruct((_NW, 2 * _NUM_BINS), jnp.float32),
        mesh=mesh,
        compiler_params=pltpu.CompilerParams(needs_layout_passes=False),
        scratch_types=[
            pltpu.VMEM((per,), jnp.int32),
            pltpu.VMEM((per,), jnp.int32),
            pltpu.VMEM((2 * _NUM_BINS * _L,), jnp.float32),
            pltpu.VMEM((2 * _NUM_BINS,), jnp.float32),
            pltpu.SemaphoreType.DMA,
            pltpu.SemaphoreType.DMA,
        ],
    )
    def hist_kernel(v_hbm, a_hbm, out_hbm, vbuf, abuf, hist, merged,
                    sem0, sem1):
        cid = lax.axis_index("c")
        sid = lax.axis_index("s")
        wid = sid * _NC + cid
        base = wid * per
        cp_v = pltpu.async_copy(v_hbm.at[pl.ds(base, per)], vbuf, sem0)
        cp_a = pltpu.async_copy(a_hbm.at[pl.ds(base, per)], abuf, sem1)

        zeros = jnp.zeros((_L,), jnp.float32)
        lane = lax.iota(jnp.int32, 16)

        def zbody(k, carry):
            for j in range(16):
                hist[pl.ds((k * 16 + j) * _L, _L)] = zeros
            return carry

        lax.fori_loop(0, 2 * _NUM_BINS // 16, zbody, 0)

        ones = jnp.ones((_L,), jnp.float32)
        aoff = lane + _NUM_BINS * _L

        unroll = 16

        cp_v.wait()
        cp_a.wait()

        def body(i, carry):
            vb = i * (unroll * _L)
            vvecs = [
                (vbuf[pl.ds(vb + u * _L, _L)] << 4) | lane
                for u in range(unroll)
            ]
            avecs = [
                (abuf[pl.ds(vb + u * _L, _L)] << 4) + aoff
                for u in range(unroll)
            ]
            for u in range(unroll):
                plsc.addupdate_scatter(hist, [vvecs[u]], ones)
            for u in range(unroll):
                plsc.addupdate_scatter(hist, [avecs[u]], ones)
            return carry

        lax.fori_loop(0, (per // _L) // unroll, body, 0)

        lane16 = lane << 4

        def mbody(v, carry):
            vbase = v * (_L * _L)
            acc = jnp.zeros((_L,), jnp.float32)
            for l in range(_L):
                acc = acc + plsc.load_gather(hist, [lane16 + (vbase + l)])
            merged[pl.ds(v * _L, _L)] = acc
            return carry

        lax.fori_loop(0, 2 * _NUM_BINS // _L, mbody, 0)

        pltpu.sync_copy(merged, out_hbm.at[wid])

    return hist_kernel


def _tc_body(partial_ref, vtr_ref, atr_ref, assoc_ref, out_ref):
    counts = jnp.sum(partial_ref[...], axis=0)
    cv = counts[0:_NUM_BINS]
    ca = counts[_NUM_BINS:2 * _NUM_BINS]
    vtr = vtr_ref[...]
    atr = atr_ref[...]
    acc = (-_WD) * assoc_ref[...]
    shape = (_NUM_BINS, _NUM_BINS)
    for s in range(_N_SCALES):
        vt = _LAM_V[s] * vtr[s] + cv
        at = _LAM_A[s] * atr[s] + ca
        acc = acc + _W_SCALE * (
            lax.broadcast_in_dim(vt, shape, (0,))
            * lax.broadcast_in_dim(at, shape, (1,))
        )
    out_ref[...] = jnp.clip(acc, -_CLIP, _CLIP)


def kernel(visual_codes, audio_codes, visual_traces, audio_traces, association):
    n = visual_codes.shape[0]
    partial = _make_sc_hist(n)(visual_codes, audio_codes)
    update = pl.pallas_call(
        _tc_body,
        out_shape=jax.ShapeDtypeStruct((_NUM_BINS, _NUM_BINS), jnp.float32),
    )(partial, visual_traces, audio_traces, association)
    return update
